# baseline (speedup 1.0000x reference)
import functools

import jax
import jax.numpy as jnp
from jax import lax
from jax.experimental import pallas as pl
from jax.experimental.pallas import tpu as pltpu
from jax.experimental.pallas import tpu_sc as plsc

N, E, D = 10000, 320000, 128
DH = D // 2
KSTEPS = 3
EPS = 1e-5

NC, NS, L = 2, 16, 16
NW = NC * NS
BLK = 80
NBLK_REAL = E // NW // BLK
NBLK = 128
TRASH = N
NR = N + 16
RPT = 624
ZCH = 104

_mesh = plsc.VectorSubcoreMesh(
    core_axis_name="c", subcore_axis_name="s", num_cores=NC, num_subcores=NS
)
_sc_params = pltpu.CompilerParams(
    use_tc_tiling_on_sc=False, needs_layout_passes=False
)


def _deg_body(src_hbm, out_hbm, idx_v, hist_v):
    c = lax.axis_index("c")
    s = lax.axis_index("s")
    wid = s * NC + c

    z = jnp.zeros((L,), jnp.float32)
    for i in range(NR // L):
        hist_v[pl.ds(i * L, L)] = z

    pltpu.sync_copy(src_hbm.at[wid], idx_v)
    ones16 = jnp.ones((L,), jnp.float32)

    @pl.loop(0, NBLK)
    def _(j):
        for q in range(BLK // L):
            idx16 = idx_v[j, pl.ds(q * L, L)]
            plsc.addupdate_scatter(hist_v, [idx16], ones16)

    pltpu.sync_copy(hist_v.at[pl.ds(0, N)], out_hbm.at[wid])


_deg_kernel = functools.partial(
    pl.kernel,
    out_type=jax.ShapeDtypeStruct((NW, N), jnp.float32),
    mesh=_mesh,
    scratch_types=[
        pltpu.VMEM((NBLK, BLK), jnp.int32),
        pltpu.VMEM((NR,), jnp.float32),
    ],
    compiler_params=_sc_params,
)(_deg_body)


def _step_body(g_hbm, src_hbm, dst_hbm, out_hbm, si_v, di_v,
               iga_v, isa_v, igb_v, isb_v, igc_v, isc_v, igd_v, isd_v,
               rowsa_v, rowsb_v, rowsc_v, rowsd_v, zb_v, acc_sh,
               sema, semb, semc, semd, ssema, ssemb, ssemc, ssemd):
    c = lax.axis_index("c")
    s = lax.axis_index("s")
    wid = s * NC + c

    z = jnp.zeros((L,), jnp.float32)
    for i in range(ZCH):
        for q in range(DH // L):
            zb_v[i, pl.ds(q * L, L)] = z

    for k in range(RPT // ZCH):
        pltpu.sync_copy(zb_v, acc_sh.at[pl.ds(s * RPT + k * ZCH, ZCH)])

    @pl.when(s == NS - 1)
    def _():
        pltpu.sync_copy(zb_v.at[pl.ds(0, NR - NS * RPT)],
                        acc_sh.at[pl.ds(NS * RPT, NR - NS * RPT)])

    plsc.subcore_barrier()

    pltpu.sync_copy(src_hbm.at[wid], si_v)
    pltpu.sync_copy(dst_hbm.at[wid], di_v)

    ig = (iga_v, igb_v, igc_v, igd_v)
    isv = (isa_v, isb_v, isc_v, isd_v)
    rows = (rowsa_v, rowsb_v, rowsc_v, rowsd_v)
    sems = (sema, semb, semc, semd)

    def _copy_idx(j, k):
        for q in range(BLK // L):
            ig[k][pl.ds(q * L, L)] = si_v[j, pl.ds(q * L, L)]
            isv[k][pl.ds(q * L, L)] = di_v[j, pl.ds(q * L, L)]

    NB_ = 4
    for k in range(NB_):
        _copy_idx(k, k)
        pltpu.async_copy(g_hbm.at[ig[k]], rows[k], sems[k])

    ssems = (ssema, ssemb, ssemc, ssemd)

    @pl.loop(0, NBLK // NB_ - 1)
    def _(j):
        for k in range(NB_):
            pltpu.make_async_copy(g_hbm.at[ig[k]], rows[k], sems[k]).wait()
            pltpu.async_copy(rows[k], acc_sh.at[isv[k]], ssems[k], add=True)
        for k in range(NB_):
            pltpu.make_async_copy(rows[k], acc_sh.at[isv[k]],
                                  ssems[k]).wait()
            _copy_idx(j * NB_ + k + NB_, k)
            pltpu.async_copy(g_hbm.at[ig[k]], rows[k], sems[k])

    for k in range(NB_):
        pltpu.make_async_copy(g_hbm.at[ig[k]], rows[k], sems[k]).wait()
        pltpu.async_copy(rows[k], acc_sh.at[isv[k]], ssems[k], add=True)
    for k in range(NB_):
        pltpu.make_async_copy(rows[k], acc_sh.at[isv[k]], ssems[k]).wait()

    plsc.subcore_barrier()
    pltpu.sync_copy(
        acc_sh.at[pl.ds(s * RPT, RPT)], out_hbm.at[c, pl.ds(s * RPT, RPT)]
    )

    @pl.when(s == NS - 1)
    def _():
        pltpu.sync_copy(acc_sh.at[pl.ds(NS * RPT, N - NS * RPT)],
                        out_hbm.at[c, pl.ds(NS * RPT, N - NS * RPT)])


_step_kernel = functools.partial(
    pl.kernel,
    out_type=jax.ShapeDtypeStruct((NC, N, DH), jnp.float32),
    mesh=_mesh,
    scratch_types=[
        pltpu.VMEM((NBLK, BLK), jnp.int32),
        pltpu.VMEM((NBLK, BLK), jnp.int32),
        pltpu.VMEM((BLK,), jnp.int32),
        pltpu.VMEM((BLK,), jnp.int32),
        pltpu.VMEM((BLK,), jnp.int32),
        pltpu.VMEM((BLK,), jnp.int32),
        pltpu.VMEM((BLK,), jnp.int32),
        pltpu.VMEM((BLK,), jnp.int32),
        pltpu.VMEM((BLK,), jnp.int32),
        pltpu.VMEM((BLK,), jnp.int32),
        pltpu.VMEM((BLK, DH), jnp.float32),
        pltpu.VMEM((BLK, DH), jnp.float32),
        pltpu.VMEM((BLK, DH), jnp.float32),
        pltpu.VMEM((BLK, DH), jnp.float32),
        pltpu.VMEM((ZCH, DH), jnp.float32),
        pltpu.VMEM_SHARED((NR, DH), jnp.float32),
        pltpu.SemaphoreType.DMA,
        pltpu.SemaphoreType.DMA,
        pltpu.SemaphoreType.DMA,
        pltpu.SemaphoreType.DMA,
        pltpu.SemaphoreType.DMA,
        pltpu.SemaphoreType.DMA,
        pltpu.SemaphoreType.DMA,
        pltpu.SemaphoreType.DMA,
    ],
    compiler_params=_sc_params,
)(_step_body)


_GRID = 10
_RB = N // _GRID
GR = NR


def _prep_body(degt_ref, feat_ref, g0a_ref, g0b_ref, nrm_ref, nsq_ref):
    d = jnp.sum(degt_ref[...], axis=1, keepdims=True)
    nrm = lax.rsqrt(jnp.maximum(d, 1.0))
    nrm_ref[...] = nrm
    nsq_ref[...] = nrm * nrm
    g0 = feat_ref[...] * nrm
    g0a_ref[...] = g0[:, :DH]
    g0b_ref[...] = g0[:, DH:]


def _prep(degt, feature):
    return pl.pallas_call(
        _prep_body,
        grid=(_GRID,),
        in_specs=[
            pl.BlockSpec((_RB, NW), lambda i: (i, 0)),
            pl.BlockSpec((_RB, D), lambda i: (i, 0)),
        ],
        out_specs=[
            pl.BlockSpec((_RB, DH), lambda i: (i, 0)),
            pl.BlockSpec((_RB, DH), lambda i: (i, 0)),
            pl.BlockSpec((_RB, 1), lambda i: (i, 0)),
            pl.BlockSpec((_RB, 1), lambda i: (i, 0)),
        ],
        out_shape=[
            jax.ShapeDtypeStruct((GR, DH), jnp.float32),
            jax.ShapeDtypeStruct((GR, DH), jnp.float32),
            jax.ShapeDtypeStruct((N, 1), jnp.float32),
            jax.ShapeDtypeStruct((N, 1), jnp.float32),
        ],
    )(degt, feature)


def _acc_full(pa_ref, pb_ref):
    return jnp.concatenate(
        [pa_ref[0] + pa_ref[1], pb_ref[0] + pb_ref[1]], axis=1
    )


def _combine_body(pa_ref, pb_ref, nrm_ref, nsq_ref, sin_ref,
                  sout_ref, ga_ref, gb_ref):
    acc = _acc_full(pa_ref, pb_ref)
    sout_ref[...] = sin_ref[...] + acc * nrm_ref[...]
    g = acc * nsq_ref[...]
    ga_ref[...] = g[:, :DH]
    gb_ref[...] = g[:, DH:]


def _combine(pa, pb, nrm, nsq, s_in):
    return pl.pallas_call(
        _combine_body,
        grid=(_GRID,),
        in_specs=[
            pl.BlockSpec((NC, _RB, DH), lambda i: (0, i, 0)),
            pl.BlockSpec((NC, _RB, DH), lambda i: (0, i, 0)),
            pl.BlockSpec((_RB, 1), lambda i: (i, 0)),
            pl.BlockSpec((_RB, 1), lambda i: (i, 0)),
            pl.BlockSpec((_RB, D), lambda i: (i, 0)),
        ],
        out_specs=[
            pl.BlockSpec((_RB, D), lambda i: (i, 0)),
            pl.BlockSpec((_RB, DH), lambda i: (i, 0)),
            pl.BlockSpec((_RB, DH), lambda i: (i, 0)),
        ],
        out_shape=[
            jax.ShapeDtypeStruct((N, D), jnp.float32),
            jax.ShapeDtypeStruct((GR, DH), jnp.float32),
            jax.ShapeDtypeStruct((GR, DH), jnp.float32),
        ],
    )(pa, pb, nrm, nsq, s_in)


def _final_mm_body(feat_ref, s_ref, pa_ref, pb_ref, nrm_ref, w_ref, b_ref,
                   z_ref, cs_ref, cq_ref):
    i = pl.program_id(0)
    x = feat_ref[...] + s_ref[...] + _acc_full(pa_ref, pb_ref) * nrm_ref[...]
    z = jnp.dot(x, w_ref[...], preferred_element_type=jnp.float32)
    z = z + (KSTEPS + 1) * b_ref[...]
    z_ref[...] = z

    @pl.when(i == 0)
    def _():
        cs_ref[...] = jnp.zeros_like(cs_ref)
        cq_ref[...] = jnp.zeros_like(cq_ref)

    cs_ref[...] += jnp.sum(z, axis=0, keepdims=True)
    cq_ref[...] += jnp.sum(z * z, axis=0, keepdims=True)


def _final_mm(feature, s_in, pa, pb, nrm, w, b2d):
    return pl.pallas_call(
        _final_mm_body,
        grid=(_GRID,),
        in_specs=[
            pl.BlockSpec((_RB, D), lambda i: (i, 0)),
            pl.BlockSpec((_RB, D), lambda i: (i, 0)),
            pl.BlockSpec((NC, _RB, DH), lambda i: (0, i, 0)),
            pl.BlockSpec((NC, _RB, DH), lambda i: (0, i, 0)),
            pl.BlockSpec((_RB, 1), lambda i: (i, 0)),
            pl.BlockSpec((D, D), lambda i: (0, 0)),
            pl.BlockSpec((1, D), lambda i: (0, 0)),
        ],
        out_specs=[
            pl.BlockSpec((_RB, D), lambda i: (i, 0)),
            pl.BlockSpec((1, D), lambda i: (0, 0)),
            pl.BlockSpec((1, D), lambda i: (0, 0)),
        ],
        out_shape=[
            jax.ShapeDtypeStruct((N, D), jnp.float32),
            jax.ShapeDtypeStruct((1, D), jnp.float32),
            jax.ShapeDtypeStruct((1, D), jnp.float32),
        ],
    )(feature, s_in, pa, pb, nrm, w, b2d)


def _final_bn_body(z_ref, cs_ref, cq_ref, gamma_ref, beta_ref, out_ref):
    mean = cs_ref[...] * (1.0 / N)
    var = cq_ref[...] * (1.0 / N) - mean * mean
    scale = lax.rsqrt(var + EPS) * gamma_ref[...]
    out_ref[...] = jnp.maximum((z_ref[...] - mean) * scale + beta_ref[...], 0.0)


def _final_bn(z, cs, cq, gamma2d, beta2d):
    return pl.pallas_call(
        _final_bn_body,
        grid=(_GRID,),
        in_specs=[
            pl.BlockSpec((_RB, D), lambda i: (i, 0)),
            pl.BlockSpec((1, D), lambda i: (0, 0)),
            pl.BlockSpec((1, D), lambda i: (0, 0)),
            pl.BlockSpec((1, D), lambda i: (0, 0)),
            pl.BlockSpec((1, D), lambda i: (0, 0)),
        ],
        out_specs=pl.BlockSpec((_RB, D), lambda i: (i, 0)),
        out_shape=jax.ShapeDtypeStruct((N, D), jnp.float32),
    )(z, cs, cq, gamma2d, beta2d)


@jax.jit
def kernel(feature, edge_index, W, b, gamma, beta):
    pad = jnp.full((NW, NBLK - NBLK_REAL, BLK), TRASH, jnp.int32)
    src = jnp.concatenate(
        [edge_index[0].reshape(NW, NBLK_REAL, BLK), pad], axis=1)
    dst = jnp.concatenate(
        [edge_index[1].reshape(NW, NBLK_REAL, BLK), pad], axis=1)

    deg = _deg_kernel(src)
    ga, gb, nrm, nsq = _prep(deg.T, feature)

    s = jnp.zeros((N, D), jnp.float32)
    for _ in range(KSTEPS - 1):
        pa = _step_kernel(ga, src, dst)
        pb = _step_kernel(gb, src, dst)
        s, ga, gb = _combine(pa, pb, nrm, nsq, s)
    pa = _step_kernel(ga, src, dst)
    pb = _step_kernel(gb, src, dst)

    z, cs, cq = _final_mm(feature, s, pa, pb, nrm, W, b.reshape(1, D))
    return _final_bn(z, cs, cq, gamma.reshape(1, D), beta.reshape(1, D))


# speedup vs baseline: 1.0400x; 1.0400x over previous
import functools

import jax
import jax.numpy as jnp
from jax import lax
from jax.experimental import pallas as pl
from jax.experimental.pallas import tpu as pltpu
from jax.experimental.pallas import tpu_sc as plsc

N, E, D = 10000, 320000, 128
DH = D // 2
KSTEPS = 3
EPS = 1e-5

NC, NS, L = 2, 16, 16
NW = NC * NS
BLK = 80
NBLK_REAL = E // NW // BLK
NBLK = 128
TRASH = N
NR = N + 16
RPT = 624
ZCH = 104

_mesh = plsc.VectorSubcoreMesh(
    core_axis_name="c", subcore_axis_name="s", num_cores=NC, num_subcores=NS
)
_sc_params = pltpu.CompilerParams(
    use_tc_tiling_on_sc=False, needs_layout_passes=False
)


def _deg_body(src_hbm, out_hbm, idx_v, hist_v):
    c = lax.axis_index("c")
    s = lax.axis_index("s")
    wid = s * NC + c

    z = jnp.zeros((L,), jnp.float32)
    for i in range(NR // L):
        hist_v[pl.ds(i * L, L)] = z

    pltpu.sync_copy(src_hbm.at[wid], idx_v)
    ones16 = jnp.ones((L,), jnp.float32)

    @pl.loop(0, NBLK)
    def _(j):
        for q in range(BLK // L):
            idx16 = idx_v[j, pl.ds(q * L, L)]
            plsc.addupdate_scatter(hist_v, [idx16], ones16)

    pltpu.sync_copy(hist_v.at[pl.ds(0, N)], out_hbm.at[wid])


_deg_kernel = functools.partial(
    pl.kernel,
    out_type=jax.ShapeDtypeStruct((NW, N), jnp.float32),
    mesh=_mesh,
    scratch_types=[
        pltpu.VMEM((NBLK, BLK), jnp.int32),
        pltpu.VMEM((NR,), jnp.float32),
    ],
    compiler_params=_sc_params,
)(_deg_body)


def _step_body(g_hbm, src_hbm, dst_hbm, out_hbm, si_v, di_v,
               iga_v, isa_v, igb_v, isb_v, igc_v, isc_v, igd_v, isd_v,
               rowsa_v, rowsb_v, rowsc_v, rowsd_v, zb_v, acc_sh,
               sema, semb, semc, semd):
    c = lax.axis_index("c")
    s = lax.axis_index("s")
    wid = s * NC + c

    z = jnp.zeros((L,), jnp.float32)
    for i in range(ZCH):
        for q in range(DH // L):
            zb_v[i, pl.ds(q * L, L)] = z

    for k in range(RPT // ZCH):
        pltpu.sync_copy(zb_v, acc_sh.at[pl.ds(s * RPT + k * ZCH, ZCH)])

    @pl.when(s == NS - 1)
    def _():
        pltpu.sync_copy(zb_v.at[pl.ds(0, NR - NS * RPT)],
                        acc_sh.at[pl.ds(NS * RPT, NR - NS * RPT)])

    plsc.subcore_barrier()

    pltpu.sync_copy(src_hbm.at[wid], si_v)
    pltpu.sync_copy(dst_hbm.at[wid], di_v)

    ig = (iga_v, igb_v, igc_v, igd_v)
    isv = (isa_v, isb_v, isc_v, isd_v)
    rows = (rowsa_v, rowsb_v, rowsc_v, rowsd_v)
    sems = (sema, semb, semc, semd)

    def _copy_idx(j, k):
        for q in range(BLK // L):
            ig[k][pl.ds(q * L, L)] = si_v[j, pl.ds(q * L, L)]
            isv[k][pl.ds(q * L, L)] = di_v[j, pl.ds(q * L, L)]

    NB_ = 4
    for k in range(NB_):
        _copy_idx(k, k)
        pltpu.async_copy(g_hbm.at[ig[k]], rows[k], sems[k])

    @pl.loop(0, NBLK // NB_ - 1)
    def _(j):
        for k in range(NB_):
            pltpu.make_async_copy(g_hbm.at[ig[k]], rows[k], sems[k]).wait()
            pltpu.sync_copy(rows[k], acc_sh.at[isv[k]], add=True)
            _copy_idx(j * NB_ + k + NB_, k)
            pltpu.async_copy(g_hbm.at[ig[k]], rows[k], sems[k])

    for k in range(NB_):
        pltpu.make_async_copy(g_hbm.at[ig[k]], rows[k], sems[k]).wait()
        pltpu.sync_copy(rows[k], acc_sh.at[isv[k]], add=True)

    plsc.subcore_barrier()
    pltpu.sync_copy(
        acc_sh.at[pl.ds(s * RPT, RPT)], out_hbm.at[c, pl.ds(s * RPT, RPT)]
    )

    @pl.when(s == NS - 1)
    def _():
        pltpu.sync_copy(acc_sh.at[pl.ds(NS * RPT, N - NS * RPT)],
                        out_hbm.at[c, pl.ds(NS * RPT, N - NS * RPT)])


_step_kernel = functools.partial(
    pl.kernel,
    out_type=jax.ShapeDtypeStruct((NC, N, DH), jnp.float32),
    mesh=_mesh,
    scratch_types=[
        pltpu.VMEM((NBLK, BLK), jnp.int32),
        pltpu.VMEM((NBLK, BLK), jnp.int32),
        pltpu.VMEM((BLK,), jnp.int32),
        pltpu.VMEM((BLK,), jnp.int32),
        pltpu.VMEM((BLK,), jnp.int32),
        pltpu.VMEM((BLK,), jnp.int32),
        pltpu.VMEM((BLK,), jnp.int32),
        pltpu.VMEM((BLK,), jnp.int32),
        pltpu.VMEM((BLK,), jnp.int32),
        pltpu.VMEM((BLK,), jnp.int32),
        pltpu.VMEM((BLK, DH), jnp.float32),
        pltpu.VMEM((BLK, DH), jnp.float32),
        pltpu.VMEM((BLK, DH), jnp.float32),
        pltpu.VMEM((BLK, DH), jnp.float32),
        pltpu.VMEM((ZCH, DH), jnp.float32),
        pltpu.VMEM_SHARED((NR, DH), jnp.float32),
        pltpu.SemaphoreType.DMA,
        pltpu.SemaphoreType.DMA,
        pltpu.SemaphoreType.DMA,
        pltpu.SemaphoreType.DMA,
    ],
    compiler_params=_sc_params,
)(_step_body)


_GRID = 10
_RB = N // _GRID
GR = NR


def _prep_body(degt_ref, feat_ref, g0a_ref, g0b_ref, nrm_ref, nsq_ref):
    d = jnp.sum(degt_ref[...], axis=1, keepdims=True)
    nrm = lax.rsqrt(jnp.maximum(d, 1.0))
    nrm_ref[...] = nrm
    nsq_ref[...] = nrm * nrm
    g0 = feat_ref[...] * nrm
    g0a_ref[...] = g0[:, :DH]
    g0b_ref[...] = g0[:, DH:]


def _prep(degt, feature):
    return pl.pallas_call(
        _prep_body,
        grid=(_GRID,),
        in_specs=[
            pl.BlockSpec((_RB, NW), lambda i: (i, 0)),
            pl.BlockSpec((_RB, D), lambda i: (i, 0)),
        ],
        out_specs=[
            pl.BlockSpec((_RB, DH), lambda i: (i, 0)),
            pl.BlockSpec((_RB, DH), lambda i: (i, 0)),
            pl.BlockSpec((_RB, 1), lambda i: (i, 0)),
            pl.BlockSpec((_RB, 1), lambda i: (i, 0)),
        ],
        out_shape=[
            jax.ShapeDtypeStruct((GR, DH), jnp.float32),
            jax.ShapeDtypeStruct((GR, DH), jnp.float32),
            jax.ShapeDtypeStruct((N, 1), jnp.float32),
            jax.ShapeDtypeStruct((N, 1), jnp.float32),
        ],
    )(degt, feature)


def _acc_full(pa_ref, pb_ref):
    return jnp.concatenate(
        [pa_ref[0] + pa_ref[1], pb_ref[0] + pb_ref[1]], axis=1
    )


def _combine_body(pa_ref, pb_ref, nrm_ref, nsq_ref, sin_ref,
                  sout_ref, ga_ref, gb_ref):
    acc = _acc_full(pa_ref, pb_ref)
    sout_ref[...] = sin_ref[...] + acc * nrm_ref[...]
    g = acc * nsq_ref[...]
    ga_ref[...] = g[:, :DH]
    gb_ref[...] = g[:, DH:]


def _combine(pa, pb, nrm, nsq, s_in):
    return pl.pallas_call(
        _combine_body,
        grid=(_GRID,),
        in_specs=[
            pl.BlockSpec((NC, _RB, DH), lambda i: (0, i, 0)),
            pl.BlockSpec((NC, _RB, DH), lambda i: (0, i, 0)),
            pl.BlockSpec((_RB, 1), lambda i: (i, 0)),
            pl.BlockSpec((_RB, 1), lambda i: (i, 0)),
            pl.BlockSpec((_RB, D), lambda i: (i, 0)),
        ],
        out_specs=[
            pl.BlockSpec((_RB, D), lambda i: (i, 0)),
            pl.BlockSpec((_RB, DH), lambda i: (i, 0)),
            pl.BlockSpec((_RB, DH), lambda i: (i, 0)),
        ],
        out_shape=[
            jax.ShapeDtypeStruct((N, D), jnp.float32),
            jax.ShapeDtypeStruct((GR, DH), jnp.float32),
            jax.ShapeDtypeStruct((GR, DH), jnp.float32),
        ],
    )(pa, pb, nrm, nsq, s_in)


def _final_mm_body(feat_ref, s_ref, pa_ref, pb_ref, nrm_ref, w_ref, b_ref,
                   z_ref, cs_ref, cq_ref):
    i = pl.program_id(0)
    x = feat_ref[...] + s_ref[...] + _acc_full(pa_ref, pb_ref) * nrm_ref[...]
    z = jnp.dot(x, w_ref[...], preferred_element_type=jnp.float32)
    z = z + (KSTEPS + 1) * b_ref[...]
    z_ref[...] = z

    @pl.when(i == 0)
    def _():
        cs_ref[...] = jnp.zeros_like(cs_ref)
        cq_ref[...] = jnp.zeros_like(cq_ref)

    cs_ref[...] += jnp.sum(z, axis=0, keepdims=True)
    cq_ref[...] += jnp.sum(z * z, axis=0, keepdims=True)


def _final_mm(feature, s_in, pa, pb, nrm, w, b2d):
    return pl.pallas_call(
        _final_mm_body,
        grid=(_GRID,),
        in_specs=[
            pl.BlockSpec((_RB, D), lambda i: (i, 0)),
            pl.BlockSpec((_RB, D), lambda i: (i, 0)),
            pl.BlockSpec((NC, _RB, DH), lambda i: (0, i, 0)),
            pl.BlockSpec((NC, _RB, DH), lambda i: (0, i, 0)),
            pl.BlockSpec((_RB, 1), lambda i: (i, 0)),
            pl.BlockSpec((D, D), lambda i: (0, 0)),
            pl.BlockSpec((1, D), lambda i: (0, 0)),
        ],
        out_specs=[
            pl.BlockSpec((_RB, D), lambda i: (i, 0)),
            pl.BlockSpec((1, D), lambda i: (0, 0)),
            pl.BlockSpec((1, D), lambda i: (0, 0)),
        ],
        out_shape=[
            jax.ShapeDtypeStruct((N, D), jnp.float32),
            jax.ShapeDtypeStruct((1, D), jnp.float32),
            jax.ShapeDtypeStruct((1, D), jnp.float32),
        ],
    )(feature, s_in, pa, pb, nrm, w, b2d)


def _final_bn_body(z_ref, cs_ref, cq_ref, gamma_ref, beta_ref, out_ref):
    mean = cs_ref[...] * (1.0 / N)
    var = cq_ref[...] * (1.0 / N) - mean * mean
    scale = lax.rsqrt(var + EPS) * gamma_ref[...]
    out_ref[...] = jnp.maximum((z_ref[...] - mean) * scale + beta_ref[...], 0.0)


def _final_bn(z, cs, cq, gamma2d, beta2d):
    return pl.pallas_call(
        _final_bn_body,
        grid=(_GRID,),
        in_specs=[
            pl.BlockSpec((_RB, D), lambda i: (i, 0)),
            pl.BlockSpec((1, D), lambda i: (0, 0)),
            pl.BlockSpec((1, D), lambda i: (0, 0)),
            pl.BlockSpec((1, D), lambda i: (0, 0)),
            pl.BlockSpec((1, D), lambda i: (0, 0)),
        ],
        out_specs=pl.BlockSpec((_RB, D), lambda i: (i, 0)),
        out_shape=jax.ShapeDtypeStruct((N, D), jnp.float32),
    )(z, cs, cq, gamma2d, beta2d)


@jax.jit
def kernel(feature, edge_index, W, b, gamma, beta):
    pad = jnp.full((NW, NBLK - NBLK_REAL, BLK), TRASH, jnp.int32)
    src = jnp.concatenate(
        [edge_index[0].reshape(NW, NBLK_REAL, BLK), pad], axis=1)
    dst = jnp.concatenate(
        [edge_index[1].reshape(NW, NBLK_REAL, BLK), pad], axis=1)

    deg = _deg_kernel(src)
    ga, gb, nrm, nsq = _prep(deg.T, feature)

    s = jnp.zeros((N, D), jnp.float32)
    for _ in range(KSTEPS - 1):
        pa = _step_kernel(ga, src, dst)
        pb = _step_kernel(gb, src, dst)
        s, ga, gb = _combine(pa, pb, nrm, nsq, s)
    pa = _step_kernel(ga, src, dst)
    pb = _step_kernel(gb, src, dst)

    z, cs, cq = _final_mm(feature, s, pa, pb, nrm, W, b.reshape(1, D))
    return _final_bn(z, cs, cq, gamma.reshape(1, D), beta.reshape(1, D))
